# baseline (speedup 1.0000x reference)
import functools

import jax
import jax.numpy as jnp
from jax import lax
from jax.experimental import pallas as pl
from jax.experimental.pallas import tpu as pltpu
from jax.experimental.pallas import tpu_sc as plsc

N = 10000
D = 128
E_R = 320000
E_T = 160000
NC, NS, L = 2, 16, 16
NW = NC * NS
CH = 128
EP_R = 327680
EP_T = 163840
B = 256
NP = 10112
RPS = NP // NS

_sc_cache = {}


def _mesh():
    return plsc.VectorSubcoreMesh(core_axis_name="c", subcore_axis_name="s",
                                  num_cores=NC, num_subcores=NS)


def _gather2(E, EP):
    key = ("g2", E, EP)
    if key in _sc_cache:
        return _sc_cache[key]
    per_w = EP // NW
    n_ch = per_w // CH

    @functools.partial(
        pl.kernel,
        out_type=(jax.ShapeDtypeStruct((EP, D), jnp.float32),
                  jax.ShapeDtypeStruct((EP, D), jnp.float32)),
        mesh=_mesh(),
        scratch_types=(pltpu.VMEM((CH,), jnp.int32),
                       pltpu.VMEM((CH, D), jnp.float32),
                       pltpu.VMEM((CH,), jnp.int32),
                       pltpu.VMEM((CH, D), jnp.float32),
                       pltpu.SemaphoreType.DMA,
                       pltpu.SemaphoreType.DMA))
    def k(ta, ia, tb, ib, oa, ob, iva, rva, ivb, rvb, sa, sb):
        wid = lax.axis_index("s") * NC + lax.axis_index("c")

        def body(c, carry):
            base = wid * per_w + c * CH

            @pl.when(base < E)
            def _():
                pltpu.sync_copy(ia.at[pl.ds(base, CH)], iva)
                pltpu.sync_copy(ib.at[pl.ds(base, CH)], ivb)
                da = pltpu.async_copy(ta.at[iva], rva, sa)
                db = pltpu.async_copy(tb.at[ivb], rvb, sb)
                da.wait()
                db.wait()
                pltpu.sync_copy(rva, oa.at[pl.ds(base, CH)])
                pltpu.sync_copy(rvb, ob.at[pl.ds(base, CH)])
            return carry

        lax.fori_loop(0, n_ch, body, 0)

    _sc_cache[key] = k
    return k


def _gather4(E, EP):
    key = ("g4", E, EP)
    if key in _sc_cache:
        return _sc_cache[key]
    per_w = EP // NW
    n_ch = per_w // CH

    @functools.partial(
        pl.kernel,
        out_type=tuple(jax.ShapeDtypeStruct((EP, D), jnp.float32)
                       for _ in range(4)),
        mesh=_mesh(),
        scratch_types=(tuple(pltpu.VMEM((CH,), jnp.int32) for _ in range(4))
                       + tuple(pltpu.VMEM((CH, D), jnp.float32)
                               for _ in range(4))
                       + tuple(pltpu.SemaphoreType.DMA for _ in range(4))))
    def k(t0, i0, t1, i1, t2, i2, t3, i3, o0, o1, o2, o3,
          v0, v1, v2, v3, r0, r1, r2, r3, s0, s1, s2, s3):
        wid = lax.axis_index("s") * NC + lax.axis_index("c")
        ts = (t0, t1, t2, t3)
        iss = (i0, i1, i2, i3)
        os_ = (o0, o1, o2, o3)
        vs = (v0, v1, v2, v3)
        rs = (r0, r1, r2, r3)
        ss = (s0, s1, s2, s3)

        def body(c, carry):
            base = wid * per_w + c * CH

            @pl.when(base < E)
            def _():
                for j in range(4):
                    pltpu.sync_copy(iss[j].at[pl.ds(base, CH)], vs[j])
                ds = [pltpu.async_copy(ts[j].at[vs[j]], rs[j], ss[j])
                      for j in range(4)]
                for j in range(4):
                    ds[j].wait()
                for j in range(4):
                    pltpu.sync_copy(rs[j], os_[j].at[pl.ds(base, CH)])
            return carry

        lax.fori_loop(0, n_ch, body, 0)

    _sc_cache[key] = k
    return k


def _gather_scatter(E, EP):
    key = ("gs", E, EP)
    if key in _sc_cache:
        return _sc_cache[key]
    per_w = EP // NW
    n_ch = per_w // CH

    @functools.partial(
        pl.kernel,
        out_type=jax.ShapeDtypeStruct((NC * NP, D), jnp.float32),
        mesh=_mesh(),
        scratch_types=(pltpu.VMEM((CH,), jnp.int32),
                       pltpu.VMEM((CH,), jnp.int32),
                       pltpu.VMEM((CH, D), jnp.float32),
                       pltpu.VMEM_SHARED((NP, D), jnp.float32),
                       pltpu.SemaphoreType.DMA))
    def k(ta, src_h, dst_h, zz_h, out_h, ivs, ivd, rv, acc, sem):
        cid = lax.axis_index("c")
        sid = lax.axis_index("s")
        wid = sid * NC + cid
        pltpu.sync_copy(zz_h.at[pl.ds(sid * RPS, RPS)],
                        acc.at[pl.ds(sid * RPS, RPS)])
        plsc.subcore_barrier()

        def body(c, carry):
            base = wid * per_w + c * CH

            @pl.when(base < E)
            def _():
                pltpu.sync_copy(src_h.at[pl.ds(base, CH)], ivs)
                pltpu.sync_copy(dst_h.at[pl.ds(base, CH)], ivd)
                pltpu.async_copy(ta.at[ivs], rv, sem).wait()
                pltpu.sync_copy(rv, acc.at[ivd], add=True)
            return carry

        lax.fori_loop(0, n_ch, body, 0)
        plsc.subcore_barrier()
        pltpu.sync_copy(acc.at[pl.ds(sid * RPS, RPS)],
                        out_h.at[pl.ds(cid * NP + sid * RPS, RPS)])

    _sc_cache[key] = k
    return k


def _scatter_fused(E, EP):
    key = ("sf", E, EP)
    if key in _sc_cache:
        return _sc_cache[key]
    per_w = EP // NW
    n_ch = per_w // CH

    @functools.partial(
        pl.kernel,
        out_type=(jax.ShapeDtypeStruct((NC * NP, D), jnp.float32),
                  jax.ShapeDtypeStruct((NW * NP,), jnp.float32)),
        mesh=_mesh(),
        compiler_params=pltpu.CompilerParams(needs_layout_passes=False),
        scratch_types=(pltpu.VMEM((CH,), jnp.int32),
                       pltpu.VMEM((CH, D), jnp.float32),
                       pltpu.VMEM((CH,), jnp.float32),
                       pltpu.VMEM((NP,), jnp.float32),
                       pltpu.VMEM_SHARED((NP, D), jnp.float32)))
    def k(rows_h, ex_h, idx_h, zz_h, num_h, sp_h, iv, rv, ev, accs, acc):
        cid = lax.axis_index("c")
        sid = lax.axis_index("s")
        wid = sid * NC + cid
        pltpu.sync_copy(zz_h.at[pl.ds(sid * RPS, RPS)],
                        acc.at[pl.ds(sid * RPS, RPS)])
        z16 = jnp.zeros((L,), jnp.float32)

        def zb(i, carry):
            accs[pl.ds(i * L, L)] = z16
            return carry

        lax.fori_loop(0, NP // L, zb, 0)
        plsc.subcore_barrier()

        def body(c, carry):
            base = wid * per_w + c * CH

            @pl.when(base < E)
            def _():
                pltpu.sync_copy(idx_h.at[pl.ds(base, CH)], iv)
                pltpu.sync_copy(rows_h.at[pl.ds(base, CH)], rv)
                pltpu.sync_copy(ex_h.at[pl.ds(base, CH)], ev)
                pltpu.sync_copy(rv, acc.at[iv], add=True)
                for g in range(CH // L):
                    plsc.addupdate_scatter(accs, [iv[pl.ds(g * L, L)]],
                                           ev[pl.ds(g * L, L)])
            return carry

        lax.fori_loop(0, n_ch, body, 0)
        plsc.subcore_barrier()
        pltpu.sync_copy(acc.at[pl.ds(sid * RPS, RPS)],
                        num_h.at[pl.ds(cid * NP + sid * RPS, RPS)])
        pltpu.sync_copy(accs, sp_h.at[pl.ds(wid * NP, NP)])

    _sc_cache[key] = k
    return k


def _scatter_scalar(E, EP):
    key = ("ss", E, EP)
    if key in _sc_cache:
        return _sc_cache[key]
    per_w = EP // NW
    n_g = per_w // L

    @functools.partial(
        pl.kernel,
        out_type=jax.ShapeDtypeStruct((NW * NP,), jnp.float32),
        mesh=_mesh(),
        compiler_params=pltpu.CompilerParams(needs_layout_passes=False),
        scratch_types=(pltpu.VMEM((NP,), jnp.float32),
                       pltpu.VMEM((per_w,), jnp.int32),
                       pltpu.VMEM((per_w,), jnp.float32)))
    def k(vals_h, idx_h, out_h, acc, iv, vv):
        cid = lax.axis_index("c")
        sid = lax.axis_index("s")
        wid = sid * NC + cid
        zz = jnp.zeros((L,), jnp.float32)

        def zb(i, carry):
            acc[pl.ds(i * L, L)] = zz
            return carry

        lax.fori_loop(0, NP // L, zb, 0)
        pltpu.sync_copy(idx_h.at[pl.ds(wid * per_w, per_w)], iv)
        pltpu.sync_copy(vals_h.at[pl.ds(wid * per_w, per_w)], vv)

        def body(g, carry):
            @pl.when(wid * per_w + g * L < E)
            def _():
                plsc.addupdate_scatter(acc, [iv[pl.ds(g * L, L)]],
                                       vv[pl.ds(g * L, L)])
            return carry

        lax.fori_loop(0, n_g, body, 0)
        pltpu.sync_copy(acc, out_h.at[pl.ds(wid * NP, NP)])

    _sc_cache[key] = k
    return k


def _tc(fn, out_shape, *args):
    return pl.pallas_call(fn, out_shape=out_shape)(*args)


def _bn_in(x, g, b):
    mu = jnp.mean(x, axis=0, keepdims=True)
    var = jnp.mean((x - mu) * (x - mu), axis=0, keepdims=True)
    return g[None, :] * (x - mu) / jnp.sqrt(var + 1e-5) + b[None, :]


def _lrelu(x):
    return jnp.where(x >= 0, x, 0.01 * x)


def _mm(a, w):
    return jnp.dot(a, w, preferred_element_type=jnp.float32)


def _k_bn2(ut, it, gub, bub, gib, bib, ue_o, ie_o):
    ue_o[...] = _bn_in(ut[...], gub[...], bub[...])
    ie_o[...] = _bn_in(it[...], gib[...], bib[...])


def _k_prep_mm(ue, ie, w1, w2, w3, w4, w5, w6,
               hs1_o, hd1_o, hs2_o, hd2_o, hd3_o, hd4_o):
    hs1_o[...] = _mm(ue[...], w1[...])
    hd1_o[...] = _mm(ie[...], w2[...])
    hs2_o[...] = _mm(ie[...], w3[...])
    hd2_o[...] = _mm(ue[...], w4[...])
    hd3_o[...] = _mm(ue[...], w5[...])
    hd4_o[...] = _mm(ue[...], w6[...])


def _exmsg_call(E, EP, HS, HD, a):
    nvalid = E // B

    def fn(hs_ref, hd_ref, a_ref, msg_ref, ex_ref):
        i = pl.program_id(0)
        valid = i < nvalid
        hs = hs_ref[...]
        h = hs + hd_ref[...]
        lr = jnp.where(h >= 0, h, 0.2 * h)
        e = jnp.sum(lr * a_ref[...][None, :], axis=1)
        ex = jnp.exp(e)
        ex = jnp.where(valid, ex, jnp.zeros_like(ex))
        msg = hs * ex[:, None]
        msg_ref[...] = jnp.where(valid, msg, jnp.zeros_like(msg))
        ex_ref[...] = ex

    return pl.pallas_call(
        fn,
        grid=(EP // B,),
        in_specs=[pl.BlockSpec((B, D), lambda i: (i, 0)),
                  pl.BlockSpec((B, D), lambda i: (i, 0)),
                  pl.BlockSpec((D,), lambda i: (0,))],
        out_specs=[pl.BlockSpec((B, D), lambda i: (i, 0)),
                   pl.BlockSpec((B,), lambda i: (i,))],
        out_shape=[jax.ShapeDtypeStruct((EP, D), jnp.float32),
                   jax.ShapeDtypeStruct((EP,), jnp.float32)])(HS, HD, a)


def _dot_call(EP, A, Bm):
    def fn(a_ref, b_ref, o_ref):
        o_ref[...] = jnp.sum(a_ref[...] * b_ref[...], axis=1)

    return pl.pallas_call(
        fn,
        grid=(EP // B,),
        in_specs=[pl.BlockSpec((B, D), lambda i: (i, 0)),
                  pl.BlockSpec((B, D), lambda i: (i, 0))],
        out_specs=pl.BlockSpec((B,), lambda i: (i,)),
        out_shape=jax.ShapeDtypeStruct((EP,), jnp.float32))(A, Bm)


def _seg_div(num_ref, sp_ref):
    num = num_ref[:N, :] + num_ref[NP:NP + N, :]
    s = jnp.sum(sp_ref[:, :N], axis=0)
    return num / (s[:, None] + 1e-9)


def _k_div_mm(num_ref, sp_ref, w_ref, o_ref):
    o_ref[...] = _mm(_seg_div(num_ref, sp_ref), w_ref[...])


def _k_norm_prep(degp_ref, ue_ref, norm_o, x0n_o):
    deg = jnp.sum(degp_ref[:, :N], axis=0)
    norm = 1.0 / jnp.sqrt(jnp.maximum(deg, 1.0))
    norm_o[...] = norm
    x0n_o[...] = ue_ref[...] * norm[:, None]


def _k_cheb1(p_ref, x0_ref, norm_ref, lam_ref, x1_o, x1n_o):
    norm = norm_ref[...]
    ax = norm[:, None] * (p_ref[:N, :] + p_ref[NP:NP + N, :])
    re = 2.0 / lam_ref[0]
    x0 = x0_ref[...]
    x1 = re * (x0 - ax) - x0
    x1_o[...] = x1
    x1n_o[...] = x1 * norm[:, None]


def _cheb2_core(p_ref, x0_ref, x1_ref, norm_ref, lam_ref, wc_ref, bc_ref):
    norm = norm_ref[...]
    ax = norm[:, None] * (p_ref[:N, :] + p_ref[NP:NP + N, :])
    re = 2.0 / lam_ref[0]
    x0 = x0_ref[...]
    x1 = x1_ref[...]
    x2 = 2.0 * (re * (x1 - ax) - x1) - x0
    return (_mm(x0, wc_ref[0]) + _mm(x1, wc_ref[1]) + _mm(x2, wc_ref[2])
            + bc_ref[...][None, :])


def _k_cheb2a(p_ref, x0_ref, x1_ref, norm_ref, lam_ref, wc_ref, bc_ref,
              y_o, yn_o):
    y = _cheb2_core(p_ref, x0_ref, x1_ref, norm_ref, lam_ref, wc_ref, bc_ref)
    y_o[...] = y
    yn_o[...] = y * norm_ref[...][:, None]


def _k_cheb2b(p_ref, x0_ref, x1_ref, norm_ref, lam_ref, wc_ref, bc_ref,
              ws_ref, wd_ref, hs_o, hd_o):
    y = _cheb2_core(p_ref, x0_ref, x1_ref, norm_ref, lam_ref, wc_ref, bc_ref)
    hs_o[...] = _mm(y, ws_ref[...])
    hd_o[...] = _mm(y, wd_ref[...])


def _k_div2(n3_ref, s3_ref, n4_ref, s4_ref, ii_o, si_o):
    ii_o[...] = _seg_div(n3_ref, s3_ref)
    si_o[...] = _seg_div(n4_ref, s4_ref)


def _k_d1(ii_ref, si_ref, ue_ref,
          wso_ref, bso_ref, gso_ref, bbso_ref,
          wco_ref, bco_ref, gco_ref, bbco_ref, o_ref):
    item_infl = ii_ref[...]
    social_item = si_ref[...]
    uie = _mm(item_infl, wso_ref[:D, :]) + _mm(social_item, wso_ref[D:, :])
    uie = _lrelu(_bn_in(uie + bso_ref[...][None, :], gso_ref[...], bbso_ref[...]))
    hup = _mm(uie, wco_ref[:D, :]) + _mm(ue_ref[...], wco_ref[D:, :])
    o_ref[...] = _lrelu(_bn_in(hup + bco_ref[...][None, :], gco_ref[...],
                               bbco_ref[...]))


def _k_d2(n5_ref, s5_ref, ue_ref, wsc_ref, bsc_ref, gsc_ref, bbsc_ref, o_ref):
    use = _seg_div(n5_ref, s5_ref)
    hus = _mm(use, wsc_ref[:D, :]) + _mm(ue_ref[...], wsc_ref[D:, :])
    o_ref[...] = _lrelu(_bn_in(hus + bsc_ref[...][None, :], gsc_ref[...],
                               bbsc_ref[...]))


def _k_d3(hup_ref, hus_ref, wmp_ref, bmp_ref, gmp_ref, bbmp_ref,
          wms_ref, bms_ref, gms_ref, bbms_ref, hp_o, hs_o):
    hup = hup_ref[...]
    hus = hus_ref[...]
    hm = hup * hus
    smp = jax.nn.softmax(hup, axis=1)
    sms = jax.nn.softmax(hus, axis=1)
    hp = _mm(hm * smp, wmp_ref[:D, :]) + _mm(hup, wmp_ref[D:, :])
    hp_o[...] = _lrelu(_bn_in(hp + bmp_ref[...][None, :], gmp_ref[...],
                              bbmp_ref[...]))
    hs = _mm(hm * sms, wms_ref[:D, :]) + _mm(hus, wms_ref[D:, :])
    hs_o[...] = _lrelu(_bn_in(hs + bms_ref[...][None, :], gms_ref[...],
                              bbms_ref[...]))


def _pad_idx(ix, EP):
    e = ix.shape[0]
    return jnp.concatenate([ix.astype(jnp.int32),
                            jnp.zeros((EP - e,), jnp.int32)])


_F = jnp.float32
_SDS = jax.ShapeDtypeStruct


def _gat_tail(E, EP, HS, HD, a, dst_p, zz):
    MSG, EX = _exmsg_call(E, EP, HS, HD, a)
    num, sp = _scatter_fused(E, EP)(MSG, EX, dst_p, zz)
    return num, sp.reshape(NW, NP)


def _gat(hs_table, hd_table, src_p, dst_p, a, E, EP, zz):
    HS, HD = _gather2(E, EP)(hs_table, src_p, hd_table, dst_p)
    return _gat_tail(E, EP, HS, HD, a, dst_p, zz)


def kernel(rate_edge_index, trust_edge_index, neg_rate_edge_index,
           neg_trust_edge_index, social_edge_index, laplacian_lambda_max,
           user_table, item_table,
           g_ubn, g_ibn, g_so, g_cons, g_soc, g_mp, g_ms,
           b_ubn, b_ibn, bb_so, bb_cons, bb_soc, bb_mp, bb_ms,
           b_so, b_cons, b_soc, b_mp, b_ms, b_cheb,
           Ws1r, Wd1r, Ws1b, Wd1b, Ws2b, Wd2b, Ws2t, Wd2t, Ws_sp, Wd_sp,
           a1r, a1b, a2b, a2t, a_sp,
           W_so, W_cons, W_soc, W_mp, W_ms, W_cheb):
    ru_p = _pad_idx(rate_edge_index[0], EP_R)
    ri_p = _pad_idx(rate_edge_index[1], EP_R)
    tu_p = _pad_idx(trust_edge_index[0], EP_T)
    tv_p = _pad_idx(trust_edge_index[1], EP_T)
    su_p = _pad_idx(social_edge_index[0], EP_T)
    sv_p = _pad_idx(social_edge_index[1], EP_T)
    nru_p = _pad_idx(neg_rate_edge_index[0], EP_R)
    nri_p = _pad_idx(neg_rate_edge_index[1], EP_R)
    ntu_p = _pad_idx(neg_trust_edge_index[0], EP_T)
    ntv_p = _pad_idx(neg_trust_edge_index[1], EP_T)
    zz = jnp.zeros((NP, D), _F)
    ones_t = jnp.ones((EP_T,), _F)
    lam = laplacian_lambda_max

    nd = _SDS((N, D), _F)
    ue_emb, ie_emb = _tc(_k_bn2, [nd] * 2, user_table, item_table,
                         g_ubn, b_ubn, g_ibn, b_ibn)
    hs1, hd1, hs2, hd2, hd3, hd4 = _tc(
        _k_prep_mm, [nd] * 6, ue_emb, ie_emb,
        Ws1r, Wd1r, Ws1b, Wd1b, Wd2b, Wd2t)

    g1s, g1d, g2s, g2d = _gather4(E_R, EP_R)(
        hs1, ru_p, hd1, ri_p, hs2, ri_p, hd2, ru_p)
    n1, s1 = _gat_tail(E_R, EP_R, g1s, g1d, a1r, ri_p, zz)
    hs3 = _tc(_k_div_mm, nd, n1, s1, Ws2b)
    n2, s2 = _gat_tail(E_R, EP_R, g2s, g2d, a1b, ru_p, zz)
    hs4 = _tc(_k_div_mm, nd, n2, s2, Ws2t)
    n3, s3 = _gat(hs3, hd3, ri_p, ru_p, a2b, E_R, EP_R, zz)
    n4, s4 = _gat(hs4, hd4, tu_p, tv_p, a2t, E_T, EP_T, zz)

    degp = _scatter_scalar(E_T, EP_T)(ones_t, sv_p).reshape(NW, NP)
    norm, x0n = _tc(_k_norm_prep, [_SDS((N,), _F), nd], degp, ue_emb)

    gs = _gather_scatter(E_T, EP_T)
    p = gs(x0n, su_p, sv_p, zz)
    x1, x1n = _tc(_k_cheb1, [nd, nd], p, ue_emb, norm, lam)
    p = gs(x1n, su_p, sv_p, zz)
    y1, y1n = _tc(_k_cheb2a, [nd, nd], p, ue_emb, x1, norm, lam, W_cheb,
                  b_cheb)
    p = gs(y1n, su_p, sv_p, zz)
    x1b, x1bn = _tc(_k_cheb1, [nd, nd], p, y1, norm, lam)
    p = gs(x1bn, su_p, sv_p, zz)
    hs5, hd5 = _tc(_k_cheb2b, [nd, nd], p, y1, x1b, norm, lam, W_cheb,
                   b_cheb, Ws_sp, Wd_sp)

    n5, s5 = _gat(hs5, hd5, su_p, sv_p, a_sp, E_T, EP_T, zz)

    item_infl, social_item = _tc(_k_div2, [nd, nd], n3, s3, n4, s4)
    h_up = _tc(_k_d1, nd, item_infl, social_item, ue_emb,
               W_so, b_so, g_so, bb_so, W_cons, b_cons, g_cons, bb_cons)
    h_us = _tc(_k_d2, nd, n5, s5, ue_emb, W_soc, b_soc, g_soc, bb_soc)
    h_new_p, h_new_s = _tc(_k_d3, [nd, nd], h_up, h_us,
                           W_mp, b_mp, g_mp, bb_mp, W_ms, b_ms, g_ms, bb_ms)

    a1, b1, a2, b2 = _gather4(E_R, EP_R)(
        h_new_p, ru_p, ie_emb, ri_p, h_new_p, nru_p, ie_emb, nri_p)
    pos_rate = _dot_call(EP_R, a1, b1)[:E_R].reshape(E_R, 1)
    neg_rate = _dot_call(EP_R, a2, b2)[:E_R].reshape(E_R, 1)
    a3, b3, a4, b4 = _gather4(E_T, EP_T)(
        h_new_s, tu_p, ue_emb, tv_p, h_new_s, ntu_p, ue_emb, ntv_p)
    pos_link = _dot_call(EP_T, a3, b3)[:E_T].reshape(E_T, 1)
    neg_link = _dot_call(EP_T, a4, b4)[:E_T].reshape(E_T, 1)

    return (pos_rate, neg_rate, pos_link, neg_link)


# speedup vs baseline: 1.1392x; 1.1392x over previous
import functools

import jax
import jax.numpy as jnp
from jax import lax
from jax.experimental import pallas as pl
from jax.experimental.pallas import tpu as pltpu
from jax.experimental.pallas import tpu_sc as plsc

N = 10000
D = 128
E_R = 320000
E_T = 160000
NC, NS, L = 2, 16, 16
NW = NC * NS
CH = 128
EP_R = 327680
EP_T = 163840
B = 256
NP = 10112
RPS = NP // NS

_sc_cache = {}


def _mesh():
    return plsc.VectorSubcoreMesh(core_axis_name="c", subcore_axis_name="s",
                                  num_cores=NC, num_subcores=NS)


def _gather2(E, EP):
    key = ("g2", E, EP)
    if key in _sc_cache:
        return _sc_cache[key]
    per_w = EP // NW
    n_ch = per_w // CH

    @functools.partial(
        pl.kernel,
        out_type=(jax.ShapeDtypeStruct((EP, D), jnp.float32),
                  jax.ShapeDtypeStruct((EP, D), jnp.float32)),
        mesh=_mesh(),
        scratch_types=(pltpu.VMEM((per_w,), jnp.int32),
                       pltpu.VMEM((per_w,), jnp.int32),
                       pltpu.VMEM((CH, D), jnp.float32),
                       pltpu.VMEM((CH, D), jnp.float32),
                       pltpu.VMEM((CH, D), jnp.float32),
                       pltpu.VMEM((CH, D), jnp.float32),
                       pltpu.SemaphoreType.DMA,
                       pltpu.SemaphoreType.DMA,
                       pltpu.SemaphoreType.DMA,
                       pltpu.SemaphoreType.DMA))
    def k(ta, ia, tb, ib, oa, ob, iva, ivb,
          rva0, rva1, rvb0, rvb1, sa0, sa1, sb0, sb1):
        wid = lax.axis_index("s") * NC + lax.axis_index("c")
        tbase = wid * per_w
        pltpu.sync_copy(ia.at[pl.ds(tbase, per_w)], iva)
        pltpu.sync_copy(ib.at[pl.ds(tbase, per_w)], ivb)
        rva = (rva0, rva1)
        rvb = (rvb0, rvb1)
        sa = (sa0, sa1)
        sb = (sb0, sb1)

        def issue(c, b):
            ok = jnp.logical_and(tbase + c * CH < E, c < n_ch)

            @pl.when(ok)
            def _():
                pltpu.async_copy(ta.at[iva.at[pl.ds(c * CH, CH)]],
                                 rva[b], sa[b])
                pltpu.async_copy(tb.at[ivb.at[pl.ds(c * CH, CH)]],
                                 rvb[b], sb[b])

        def consume(c, b):
            base = tbase + c * CH

            @pl.when(base < E)
            def _():
                pltpu.make_async_copy(ta.at[iva.at[pl.ds(c * CH, CH)]],
                                      rva[b], sa[b]).wait()
                pltpu.make_async_copy(tb.at[ivb.at[pl.ds(c * CH, CH)]],
                                      rvb[b], sb[b]).wait()
                pltpu.sync_copy(rva[b], oa.at[pl.ds(base, CH)])
                pltpu.sync_copy(rvb[b], ob.at[pl.ds(base, CH)])

        issue(0, 0)
        issue(1, 1)

        def body(g, carry):
            for b in range(2):
                c = 2 * g + b
                consume(c, b)
                issue(c + 2, b)
            return carry

        lax.fori_loop(0, n_ch // 2, body, 0)

    _sc_cache[key] = k
    return k


def _gather_scatter(E, EP):
    key = ("gs", E, EP)
    if key in _sc_cache:
        return _sc_cache[key]
    per_w = EP // NW
    n_ch = per_w // CH

    @functools.partial(
        pl.kernel,
        out_type=jax.ShapeDtypeStruct((NC * NP, D), jnp.float32),
        mesh=_mesh(),
        scratch_types=(pltpu.VMEM((per_w,), jnp.int32),
                       pltpu.VMEM((CH,), jnp.int32),
                       pltpu.VMEM((CH,), jnp.int32),
                       pltpu.VMEM((CH, D), jnp.float32),
                       pltpu.VMEM((CH, D), jnp.float32),
                       pltpu.VMEM_SHARED((NP, D), jnp.float32),
                       pltpu.SemaphoreType.DMA,
                       pltpu.SemaphoreType.DMA))
    def k(ta, src_h, dst_h, zz_h, out_h, ivs, ivd0, ivd1, rv0, rv1,
          acc, sg0, sg1):
        cid = lax.axis_index("c")
        sid = lax.axis_index("s")
        wid = sid * NC + cid
        tbase = wid * per_w
        pltpu.sync_copy(src_h.at[pl.ds(tbase, per_w)], ivs)
        pltpu.sync_copy(zz_h.at[pl.ds(sid * RPS, RPS)],
                        acc.at[pl.ds(sid * RPS, RPS)])
        plsc.subcore_barrier()
        rv = (rv0, rv1)
        ivd = (ivd0, ivd1)
        sg = (sg0, sg1)

        def issue(c, b):
            ok = jnp.logical_and(tbase + c * CH < E, c < n_ch)

            @pl.when(ok)
            def _():
                pltpu.async_copy(ta.at[ivs.at[pl.ds(c * CH, CH)]],
                                 rv[b], sg[b])

        def consume(c, b):
            base = tbase + c * CH

            @pl.when(base < E)
            def _():
                pltpu.sync_copy(dst_h.at[pl.ds(base, CH)], ivd[b])
                pltpu.make_async_copy(ta.at[ivs.at[pl.ds(c * CH, CH)]],
                                      rv[b], sg[b]).wait()
                pltpu.sync_copy(rv[b], acc.at[ivd[b]], add=True)

        issue(0, 0)
        issue(1, 1)

        def body(g, carry):
            for b in range(2):
                c = 2 * g + b
                consume(c, b)
                issue(c + 2, b)
            return carry

        lax.fori_loop(0, n_ch // 2, body, 0)
        plsc.subcore_barrier()
        pltpu.sync_copy(acc.at[pl.ds(sid * RPS, RPS)],
                        out_h.at[pl.ds(cid * NP + sid * RPS, RPS)])

    _sc_cache[key] = k
    return k


def _scatter_fused(E, EP):
    key = ("sf", E, EP)
    if key in _sc_cache:
        return _sc_cache[key]
    per_w = EP // NW
    n_ch = per_w // CH

    @functools.partial(
        pl.kernel,
        out_type=(jax.ShapeDtypeStruct((NC * NP, D), jnp.float32),
                  jax.ShapeDtypeStruct((NW * NP,), jnp.float32)),
        mesh=_mesh(),
        compiler_params=pltpu.CompilerParams(needs_layout_passes=False),
        scratch_types=(pltpu.VMEM((CH,), jnp.int32),
                       pltpu.VMEM((CH,), jnp.int32),
                       pltpu.VMEM((CH, D), jnp.float32),
                       pltpu.VMEM((CH, D), jnp.float32),
                       pltpu.VMEM((CH,), jnp.float32),
                       pltpu.VMEM((CH,), jnp.float32),
                       pltpu.VMEM((NP,), jnp.float32),
                       pltpu.VMEM_SHARED((NP, D), jnp.float32),
                       pltpu.SemaphoreType.DMA,
                       pltpu.SemaphoreType.DMA))
    def k(rows_h, ex_h, idx_h, zz_h, num_h, sp_h,
          iv0, iv1, rv0, rv1, ev0, ev1, accs, acc, sr0, sr1):
        cid = lax.axis_index("c")
        sid = lax.axis_index("s")
        wid = sid * NC + cid
        tbase = wid * per_w
        pltpu.sync_copy(zz_h.at[pl.ds(sid * RPS, RPS)],
                        acc.at[pl.ds(sid * RPS, RPS)])
        z16 = jnp.zeros((L,), jnp.float32)

        def zb(i, carry):
            accs[pl.ds(i * L, L)] = z16
            return carry

        lax.fori_loop(0, NP // L, zb, 0)
        plsc.subcore_barrier()
        iv = (iv0, iv1)
        rv = (rv0, rv1)
        ev = (ev0, ev1)
        sr = (sr0, sr1)

        def issue(c, b):
            ok = jnp.logical_and(tbase + c * CH < E, c < n_ch)

            @pl.when(ok)
            def _():
                pltpu.async_copy(rows_h.at[pl.ds(tbase + c * CH, CH)],
                                 rv[b], sr[b])

        def consume(c, b):
            base = tbase + c * CH

            @pl.when(base < E)
            def _():
                pltpu.sync_copy(idx_h.at[pl.ds(base, CH)], iv[b])
                pltpu.sync_copy(ex_h.at[pl.ds(base, CH)], ev[b])
                pltpu.make_async_copy(rows_h.at[pl.ds(base, CH)],
                                      rv[b], sr[b]).wait()
                pltpu.sync_copy(rv[b], acc.at[iv[b]], add=True)
                for g in range(CH // L):
                    plsc.addupdate_scatter(accs, [iv[b][pl.ds(g * L, L)]],
                                           ev[b][pl.ds(g * L, L)])

        issue(0, 0)
        issue(1, 1)

        def body(g, carry):
            for b in range(2):
                c = 2 * g + b
                consume(c, b)
                issue(c + 2, b)
            return carry

        lax.fori_loop(0, n_ch // 2, body, 0)
        plsc.subcore_barrier()
        pltpu.sync_copy(acc.at[pl.ds(sid * RPS, RPS)],
                        num_h.at[pl.ds(cid * NP + sid * RPS, RPS)])
        pltpu.sync_copy(accs, sp_h.at[pl.ds(wid * NP, NP)])

    _sc_cache[key] = k
    return k


def _scatter_scalar(E, EP):
    key = ("ss", E, EP)
    if key in _sc_cache:
        return _sc_cache[key]
    per_w = EP // NW
    n_g = per_w // L

    @functools.partial(
        pl.kernel,
        out_type=jax.ShapeDtypeStruct((NW * NP,), jnp.float32),
        mesh=_mesh(),
        compiler_params=pltpu.CompilerParams(needs_layout_passes=False),
        scratch_types=(pltpu.VMEM((NP,), jnp.float32),
                       pltpu.VMEM((per_w,), jnp.int32),
                       pltpu.VMEM((per_w,), jnp.float32)))
    def k(vals_h, idx_h, out_h, acc, iv, vv):
        cid = lax.axis_index("c")
        sid = lax.axis_index("s")
        wid = sid * NC + cid
        zz = jnp.zeros((L,), jnp.float32)

        def zb(i, carry):
            acc[pl.ds(i * L, L)] = zz
            return carry

        lax.fori_loop(0, NP // L, zb, 0)
        pltpu.sync_copy(idx_h.at[pl.ds(wid * per_w, per_w)], iv)
        pltpu.sync_copy(vals_h.at[pl.ds(wid * per_w, per_w)], vv)

        def body(g, carry):
            @pl.when(wid * per_w + g * L < E)
            def _():
                plsc.addupdate_scatter(acc, [iv[pl.ds(g * L, L)]],
                                       vv[pl.ds(g * L, L)])
            return carry

        lax.fori_loop(0, n_g, body, 0)
        pltpu.sync_copy(acc, out_h.at[pl.ds(wid * NP, NP)])

    _sc_cache[key] = k
    return k


def _tc(fn, out_shape, *args):
    return pl.pallas_call(fn, out_shape=out_shape)(*args)


def _bn_in(x, g, b):
    mu = jnp.mean(x, axis=0, keepdims=True)
    var = jnp.mean((x - mu) * (x - mu), axis=0, keepdims=True)
    return g[None, :] * (x - mu) / jnp.sqrt(var + 1e-5) + b[None, :]


def _lrelu(x):
    return jnp.where(x >= 0, x, 0.01 * x)


def _mm(a, w):
    return jnp.dot(a, w, preferred_element_type=jnp.float32)


def _k_bn2(ut, it, gub, bub, gib, bib, ue_o, ie_o):
    ue_o[...] = _bn_in(ut[...], gub[...], bub[...])
    ie_o[...] = _bn_in(it[...], gib[...], bib[...])


def _k_prep_mm(ue, ie, w1, w2, w3, w4, w5, w6,
               hs1_o, hd1_o, hs2_o, hd2_o, hd3_o, hd4_o):
    hs1_o[...] = _mm(ue[...], w1[...])
    hd1_o[...] = _mm(ie[...], w2[...])
    hs2_o[...] = _mm(ie[...], w3[...])
    hd2_o[...] = _mm(ue[...], w4[...])
    hd3_o[...] = _mm(ue[...], w5[...])
    hd4_o[...] = _mm(ue[...], w6[...])


def _exmsg_call(E, EP, HS, HD, a):
    nvalid = E // B

    def fn(hs_ref, hd_ref, a_ref, msg_ref, ex_ref):
        i = pl.program_id(0)
        valid = i < nvalid
        hs = hs_ref[...]
        h = hs + hd_ref[...]
        lr = jnp.where(h >= 0, h, 0.2 * h)
        e = jnp.sum(lr * a_ref[...][None, :], axis=1)
        ex = jnp.exp(e)
        ex = jnp.where(valid, ex, jnp.zeros_like(ex))
        msg = hs * ex[:, None]
        msg_ref[...] = jnp.where(valid, msg, jnp.zeros_like(msg))
        ex_ref[...] = ex

    return pl.pallas_call(
        fn,
        grid=(EP // B,),
        in_specs=[pl.BlockSpec((B, D), lambda i: (i, 0)),
                  pl.BlockSpec((B, D), lambda i: (i, 0)),
                  pl.BlockSpec((D,), lambda i: (0,))],
        out_specs=[pl.BlockSpec((B, D), lambda i: (i, 0)),
                   pl.BlockSpec((B,), lambda i: (i,))],
        out_shape=[jax.ShapeDtypeStruct((EP, D), jnp.float32),
                   jax.ShapeDtypeStruct((EP,), jnp.float32)])(HS, HD, a)


def _dot_call(EP, A, Bm):
    def fn(a_ref, b_ref, o_ref):
        o_ref[...] = jnp.sum(a_ref[...] * b_ref[...], axis=1)

    return pl.pallas_call(
        fn,
        grid=(EP // B,),
        in_specs=[pl.BlockSpec((B, D), lambda i: (i, 0)),
                  pl.BlockSpec((B, D), lambda i: (i, 0))],
        out_specs=pl.BlockSpec((B,), lambda i: (i,)),
        out_shape=jax.ShapeDtypeStruct((EP,), jnp.float32))(A, Bm)


def _seg_div(num_ref, sp_ref):
    num = num_ref[:N, :] + num_ref[NP:NP + N, :]
    s = jnp.sum(sp_ref[:, :N], axis=0)
    return num / (s[:, None] + 1e-9)


def _k_div_mm(num_ref, sp_ref, w_ref, o_ref):
    o_ref[...] = _mm(_seg_div(num_ref, sp_ref), w_ref[...])


def _k_norm_prep(degp_ref, ue_ref, norm_o, x0n_o):
    deg = jnp.sum(degp_ref[:, :N], axis=0)
    norm = 1.0 / jnp.sqrt(jnp.maximum(deg, 1.0))
    norm_o[...] = norm
    x0n_o[...] = ue_ref[...] * norm[:, None]


def _k_cheb1(p_ref, x0_ref, norm_ref, lam_ref, x1_o, x1n_o):
    norm = norm_ref[...]
    ax = norm[:, None] * (p_ref[:N, :] + p_ref[NP:NP + N, :])
    re = 2.0 / lam_ref[0]
    x0 = x0_ref[...]
    x1 = re * (x0 - ax) - x0
    x1_o[...] = x1
    x1n_o[...] = x1 * norm[:, None]


def _cheb2_core(p_ref, x0_ref, x1_ref, norm_ref, lam_ref, wc_ref, bc_ref):
    norm = norm_ref[...]
    ax = norm[:, None] * (p_ref[:N, :] + p_ref[NP:NP + N, :])
    re = 2.0 / lam_ref[0]
    x0 = x0_ref[...]
    x1 = x1_ref[...]
    x2 = 2.0 * (re * (x1 - ax) - x1) - x0
    return (_mm(x0, wc_ref[0]) + _mm(x1, wc_ref[1]) + _mm(x2, wc_ref[2])
            + bc_ref[...][None, :])


def _k_cheb2a(p_ref, x0_ref, x1_ref, norm_ref, lam_ref, wc_ref, bc_ref,
              y_o, yn_o):
    y = _cheb2_core(p_ref, x0_ref, x1_ref, norm_ref, lam_ref, wc_ref, bc_ref)
    y_o[...] = y
    yn_o[...] = y * norm_ref[...][:, None]


def _k_cheb2b(p_ref, x0_ref, x1_ref, norm_ref, lam_ref, wc_ref, bc_ref,
              ws_ref, wd_ref, hs_o, hd_o):
    y = _cheb2_core(p_ref, x0_ref, x1_ref, norm_ref, lam_ref, wc_ref, bc_ref)
    hs_o[...] = _mm(y, ws_ref[...])
    hd_o[...] = _mm(y, wd_ref[...])


def _k_div2(n3_ref, s3_ref, n4_ref, s4_ref, ii_o, si_o):
    ii_o[...] = _seg_div(n3_ref, s3_ref)
    si_o[...] = _seg_div(n4_ref, s4_ref)


def _k_d1(ii_ref, si_ref, ue_ref,
          wso_ref, bso_ref, gso_ref, bbso_ref,
          wco_ref, bco_ref, gco_ref, bbco_ref, o_ref):
    item_infl = ii_ref[...]
    social_item = si_ref[...]
    uie = _mm(item_infl, wso_ref[:D, :]) + _mm(social_item, wso_ref[D:, :])
    uie = _lrelu(_bn_in(uie + bso_ref[...][None, :], gso_ref[...], bbso_ref[...]))
    hup = _mm(uie, wco_ref[:D, :]) + _mm(ue_ref[...], wco_ref[D:, :])
    o_ref[...] = _lrelu(_bn_in(hup + bco_ref[...][None, :], gco_ref[...],
                               bbco_ref[...]))


def _k_d2(n5_ref, s5_ref, ue_ref, wsc_ref, bsc_ref, gsc_ref, bbsc_ref, o_ref):
    use = _seg_div(n5_ref, s5_ref)
    hus = _mm(use, wsc_ref[:D, :]) + _mm(ue_ref[...], wsc_ref[D:, :])
    o_ref[...] = _lrelu(_bn_in(hus + bsc_ref[...][None, :], gsc_ref[...],
                               bbsc_ref[...]))


def _k_d3(hup_ref, hus_ref, wmp_ref, bmp_ref, gmp_ref, bbmp_ref,
          wms_ref, bms_ref, gms_ref, bbms_ref, hp_o, hs_o):
    hup = hup_ref[...]
    hus = hus_ref[...]
    hm = hup * hus
    smp = jax.nn.softmax(hup, axis=1)
    sms = jax.nn.softmax(hus, axis=1)
    hp = _mm(hm * smp, wmp_ref[:D, :]) + _mm(hup, wmp_ref[D:, :])
    hp_o[...] = _lrelu(_bn_in(hp + bmp_ref[...][None, :], gmp_ref[...],
                              bbmp_ref[...]))
    hs = _mm(hm * sms, wms_ref[:D, :]) + _mm(hus, wms_ref[D:, :])
    hs_o[...] = _lrelu(_bn_in(hs + bms_ref[...][None, :], gms_ref[...],
                              bbms_ref[...]))


def _pad_idx(ix, EP):
    e = ix.shape[0]
    return jnp.concatenate([ix.astype(jnp.int32),
                            jnp.zeros((EP - e,), jnp.int32)])


_F = jnp.float32
_SDS = jax.ShapeDtypeStruct


def _gat_tail(E, EP, HS, HD, a, dst_p, zz):
    MSG, EX = _exmsg_call(E, EP, HS, HD, a)
    num, sp = _scatter_fused(E, EP)(MSG, EX, dst_p, zz)
    return num, sp.reshape(NW, NP)


def _gat(hs_table, hd_table, src_p, dst_p, a, E, EP, zz):
    HS, HD = _gather2(E, EP)(hs_table, src_p, hd_table, dst_p)
    return _gat_tail(E, EP, HS, HD, a, dst_p, zz)


def kernel(rate_edge_index, trust_edge_index, neg_rate_edge_index,
           neg_trust_edge_index, social_edge_index, laplacian_lambda_max,
           user_table, item_table,
           g_ubn, g_ibn, g_so, g_cons, g_soc, g_mp, g_ms,
           b_ubn, b_ibn, bb_so, bb_cons, bb_soc, bb_mp, bb_ms,
           b_so, b_cons, b_soc, b_mp, b_ms, b_cheb,
           Ws1r, Wd1r, Ws1b, Wd1b, Ws2b, Wd2b, Ws2t, Wd2t, Ws_sp, Wd_sp,
           a1r, a1b, a2b, a2t, a_sp,
           W_so, W_cons, W_soc, W_mp, W_ms, W_cheb):
    ru_p = _pad_idx(rate_edge_index[0], EP_R)
    ri_p = _pad_idx(rate_edge_index[1], EP_R)
    tu_p = _pad_idx(trust_edge_index[0], EP_T)
    tv_p = _pad_idx(trust_edge_index[1], EP_T)
    su_p = _pad_idx(social_edge_index[0], EP_T)
    sv_p = _pad_idx(social_edge_index[1], EP_T)
    nru_p = _pad_idx(neg_rate_edge_index[0], EP_R)
    nri_p = _pad_idx(neg_rate_edge_index[1], EP_R)
    ntu_p = _pad_idx(neg_trust_edge_index[0], EP_T)
    ntv_p = _pad_idx(neg_trust_edge_index[1], EP_T)
    zz = jnp.zeros((NP, D), _F)
    ones_t = jnp.ones((EP_T,), _F)
    lam = laplacian_lambda_max

    nd = _SDS((N, D), _F)
    ue_emb, ie_emb = _tc(_k_bn2, [nd] * 2, user_table, item_table,
                         g_ubn, b_ubn, g_ibn, b_ibn)
    hs1, hd1, hs2, hd2, hd3, hd4 = _tc(
        _k_prep_mm, [nd] * 6, ue_emb, ie_emb,
        Ws1r, Wd1r, Ws1b, Wd1b, Wd2b, Wd2t)

    n1, s1 = _gat(hs1, hd1, ru_p, ri_p, a1r, E_R, EP_R, zz)
    hs3 = _tc(_k_div_mm, nd, n1, s1, Ws2b)
    n2, s2 = _gat(hs2, hd2, ri_p, ru_p, a1b, E_R, EP_R, zz)
    hs4 = _tc(_k_div_mm, nd, n2, s2, Ws2t)
    n3, s3 = _gat(hs3, hd3, ri_p, ru_p, a2b, E_R, EP_R, zz)
    n4, s4 = _gat(hs4, hd4, tu_p, tv_p, a2t, E_T, EP_T, zz)

    degp = _scatter_scalar(E_T, EP_T)(ones_t, sv_p).reshape(NW, NP)
    norm, x0n = _tc(_k_norm_prep, [_SDS((N,), _F), nd], degp, ue_emb)

    gs = _gather_scatter(E_T, EP_T)
    p = gs(x0n, su_p, sv_p, zz)
    x1, x1n = _tc(_k_cheb1, [nd, nd], p, ue_emb, norm, lam)
    p = gs(x1n, su_p, sv_p, zz)
    y1, y1n = _tc(_k_cheb2a, [nd, nd], p, ue_emb, x1, norm, lam, W_cheb,
                  b_cheb)
    p = gs(y1n, su_p, sv_p, zz)
    x1b, x1bn = _tc(_k_cheb1, [nd, nd], p, y1, norm, lam)
    p = gs(x1bn, su_p, sv_p, zz)
    hs5, hd5 = _tc(_k_cheb2b, [nd, nd], p, y1, x1b, norm, lam, W_cheb,
                   b_cheb, Ws_sp, Wd_sp)

    n5, s5 = _gat(hs5, hd5, su_p, sv_p, a_sp, E_T, EP_T, zz)

    item_infl, social_item = _tc(_k_div2, [nd, nd], n3, s3, n4, s4)
    h_up = _tc(_k_d1, nd, item_infl, social_item, ue_emb,
               W_so, b_so, g_so, bb_so, W_cons, b_cons, g_cons, bb_cons)
    h_us = _tc(_k_d2, nd, n5, s5, ue_emb, W_soc, b_soc, g_soc, bb_soc)
    h_new_p, h_new_s = _tc(_k_d3, [nd, nd], h_up, h_us,
                           W_mp, b_mp, g_mp, bb_mp, W_ms, b_ms, g_ms, bb_ms)

    a1, b1 = _gather2(E_R, EP_R)(h_new_p, ru_p, ie_emb, ri_p)
    pos_rate = _dot_call(EP_R, a1, b1)[:E_R].reshape(E_R, 1)
    a2, b2 = _gather2(E_R, EP_R)(h_new_p, nru_p, ie_emb, nri_p)
    neg_rate = _dot_call(EP_R, a2, b2)[:E_R].reshape(E_R, 1)
    a3, b3 = _gather2(E_T, EP_T)(h_new_s, tu_p, ue_emb, tv_p)
    pos_link = _dot_call(EP_T, a3, b3)[:E_T].reshape(E_T, 1)
    a4, b4 = _gather2(E_T, EP_T)(h_new_s, ntu_p, ue_emb, ntv_p)
    neg_link = _dot_call(EP_T, a4, b4)[:E_T].reshape(E_T, 1)

    return (pos_rate, neg_rate, pos_link, neg_link)
